# baseline (speedup 1.0000x reference)
import jax
import jax.numpy as jnp
from jax.experimental import pallas as pl
from jax.experimental.pallas import tpu as pltpu

_TW = 15
_B = 16
_CH = 3
_H = 224
_W = 224


def _body(x_ref, o_ref, s_ref):
    i = pl.program_id(0)
    x = x_ref[...]

    @pl.when(i < _B)
    def _():
        masked = jnp.where(x < 0.0, jnp.inf, x)
        bmin = jnp.min(masked)
        bmax = jnp.max(x)

        @pl.when(i == 0)
        def _():
            s_ref[0] = bmin
            s_ref[1] = bmax

        @pl.when(i > 0)
        def _():
            s_ref[0] = jnp.minimum(s_ref[0], bmin)
            s_ref[1] = jnp.maximum(s_ref[1], bmax)

    @pl.when(i >= _B)
    def _():
        mmin = s_ref[0]
        gmax = s_ref[1]
        nab = mmin < jnp.inf
        img_min = jnp.where(nab, mmin, 0.0)
        mf = jnp.where(nab, 1.0 / (1.0 - img_min), 1.0)
        imax = gmax - img_min
        mf = jnp.where(imax != 0.0, 1.0 / imax, mf)

        y = ((x - img_min) * mf) * jnp.float32(_TW - 1)
        idx = jnp.ceil(y).astype(jnp.int32) + 1
        idx = jnp.where(x < 0.0, 0, idx)
        ok = (idx >= 1) & (idx <= _TW)
        sh = jnp.where(ok, _TW - idx, 0)
        word = jnp.where(ok, jnp.left_shift(jnp.int32(1), sh), 0)
        o_ref[...] = word.astype(jnp.uint16)


def kernel(img):
    words = pl.pallas_call(
        _body,
        grid=(2 * _B,),
        in_specs=[
            pl.BlockSpec(
                (1, _CH, _H, _W),
                lambda i: (jnp.where(i < _B, i, i - _B), 0, 0, 0),
            ),
        ],
        out_specs=pl.BlockSpec(
            (1, _CH, _H, _W),
            lambda i: (jnp.where(i < _B, 0, i - _B), 0, 0, 0),
        ),
        out_shape=jax.ShapeDtypeStruct((_B, _CH, _H, _W), jnp.uint16),
        scratch_shapes=[pltpu.SMEM((2,), jnp.float32)],
    )(img)
    masks = (jnp.uint16(1) << jnp.arange(_TW, dtype=jnp.uint16)).reshape(
        _TW, 1, 1, 1, 1
    )
    return (words[None] & masks) != 0
